# baseline (speedup 1.0000x reference)
import functools
import math

import jax
import jax.numpy as jnp
import numpy as np
from jax.experimental import pallas as pl

_NBINS = 9
_POOL = 7
_H = 224
_W = 224
_HC = _H // _POOL
_WC = _W // _POOL

_SIN = tuple(math.sin(k * math.pi / _NBINS) for k in range(_NBINS))
_COS = tuple(math.cos(k * math.pi / _NBINS) for k in range(_NBINS))


def _band_matrices():
    i = np.arange(_H)[:, None]
    r = np.arange(_H)[None, :]
    a_s = (2.0 * (r == i) + (r == i - 1) + (r == i + 1)).astype(np.float32)
    a_s[0, 1] += 1.0
    a_s[_H - 1, _H - 2] += 1.0
    a_d = ((r == i - 1).astype(np.float32) - (r == i + 1))
    a_d[0, :] = 0.0
    a_d[_H - 1, :] = 0.0
    return jnp.asarray(a_s, jnp.bfloat16), jnp.asarray(a_d, jnp.bfloat16)


def _pool_matrix():
    j = np.arange(_W)[:, None]
    c = np.arange(_WC)[None, :]
    return jnp.asarray((j // _POOL == c).astype(np.float32))


def _hog_body(x_ref, as_ref, ad_ref, pm_ref, pmt_ref, o_ref):
    qs = [_hog_pool(x_ref, as_ref, ad_ref, pm_ref, pmt_ref, ch)
          for ch in range(x_ref.shape[0])]
    for ch, q in enumerate(qs):
        cells = []
        acc = None
        for k in range(_NBINS):
            ck = q[k * _WC:(k + 1) * _WC]
            if k + 1 < _NBINS:
                ck = ck - q[(k + 1) * _WC:(k + 2) * _WC]
            cells.append(ck)
            sq = ck * ck
            acc = sq if acc is None else acc + sq
        inv = jnp.minimum(jax.lax.rsqrt(acc), 1e12)
        for k in range(_NBINS):
            o_ref[ch, k] = (cells[k] * inv).T


def _hog_pool(x_ref, as_ref, ad_ref, pm_ref, pmt_ref, ch):
    xb = x_ref[ch].astype(jnp.bfloat16)

    s = jnp.dot(as_ref[...], xb, preferred_element_type=jnp.float32)
    d = jnp.dot(ad_ref[...], xb, preferred_element_type=jnp.float32)

    sl = jnp.concatenate([s[:, 1:2], s[:, :-1]], axis=1)
    sr = jnp.concatenate([s[:, 1:], s[:, -2:-1]], axis=1)
    dl = jnp.concatenate([d[:, 1:2], d[:, :-1]], axis=1)
    dr = jnp.concatenate([d[:, 1:], d[:, -2:-1]], axis=1)
    gx = sl - sr
    gy = dl + 2.0 * d + dr

    norm = jnp.sqrt(gx * gx + gy * gy)

    neg = (gx < 0.0) | ((gx == 0.0) & (gy < 0.0))
    u = jnp.where(neg, -gy, gy)
    v = jnp.where(neg, -gx, gx)

    pm = pm_ref[...]
    cols = [jnp.dot(norm, pm, preferred_element_type=jnp.float32).T]
    for k in range(1, _NBINS):
        mask = (v * _COS[k] - u * _SIN[k]) >= 0.0
        t_k = jnp.where(mask, norm, 0.0)
        cols.append(jnp.dot(t_k, pm, preferred_element_type=jnp.float32).T)
    stacked = jnp.concatenate(cols, axis=0)

    return jnp.dot(stacked, pm_ref[...].astype(jnp.float32),
                   preferred_element_type=jnp.float32,
                   precision=jax.lax.Precision.HIGHEST)


_IMGS = 3


@jax.jit
def kernel(x):
    b, c, h, w = x.shape
    n = b * c
    xr = x.reshape(n, h, w)
    a_s, a_d = _band_matrices()
    pm = _pool_matrix()
    pmt = pm.T.astype(jnp.float32)
    out = pl.pallas_call(
        _hog_body,
        grid=(n // _IMGS,),
        in_specs=[
            pl.BlockSpec((_IMGS, h, w), lambda i: (i, 0, 0)),
            pl.BlockSpec((_H, _H), lambda i: (0, 0)),
            pl.BlockSpec((_H, _H), lambda i: (0, 0)),
            pl.BlockSpec((_W, _WC), lambda i: (0, 0)),
            pl.BlockSpec((_HC, _H), lambda i: (0, 0)),
        ],
        out_specs=pl.BlockSpec((_IMGS, _NBINS, _HC, _WC),
                               lambda i: (i, 0, 0, 0)),
        out_shape=jax.ShapeDtypeStruct((n, _NBINS, _HC, _WC), jnp.float32),
    )(xr, a_s, a_d, pm, pmt)
    return out.reshape(b, c, _NBINS, _HC, _WC)


# speedup vs baseline: 1.1586x; 1.1586x over previous
import functools
import math

import jax
import jax.numpy as jnp
import numpy as np
from jax.experimental import pallas as pl

_NBINS = 9
_POOL = 7
_H = 224
_W = 224
_HC = _H // _POOL
_WC = _W // _POOL

_SIN = tuple(math.sin(k * math.pi / _NBINS) for k in range(_NBINS))
_COS = tuple(math.cos(k * math.pi / _NBINS) for k in range(_NBINS))
_TAN = tuple(math.tan(k * math.pi / _NBINS) for k in range(_NBINS))


def _band_matrices():
    i = np.arange(_H)[:, None]
    r = np.arange(_H)[None, :]
    a_s = (2.0 * (r == i) + (r == i - 1) + (r == i + 1)).astype(np.float32)
    a_s[0, 1] += 1.0
    a_s[_H - 1, _H - 2] += 1.0
    a_d = ((r == i - 1).astype(np.float32) - (r == i + 1))
    a_d[0, :] = 0.0
    a_d[_H - 1, :] = 0.0
    return jnp.asarray(a_s, jnp.bfloat16), jnp.asarray(a_d, jnp.bfloat16)


def _pool_matrix():
    j = np.arange(_W)[:, None]
    c = np.arange(_WC)[None, :]
    return jnp.asarray((j // _POOL == c).astype(np.float32))


def _hog_body(x_ref, as_ref, ad_ref, pm_ref, pmt_ref, o_ref):
    qs = [_hog_pool(x_ref, as_ref, ad_ref, pm_ref, pmt_ref, ch)
          for ch in range(x_ref.shape[0])]
    for ch, q in enumerate(qs):
        cells = []
        acc = None
        for k in range(_NBINS):
            ck = q[k * _WC:(k + 1) * _WC]
            if k + 1 < _NBINS:
                ck = ck - q[(k + 1) * _WC:(k + 2) * _WC]
            cells.append(ck)
            sq = ck * ck
            acc = sq if acc is None else acc + sq
        inv = jnp.minimum(jax.lax.rsqrt(acc), 1e12)
        for k in range(_NBINS):
            o_ref[ch, k] = (cells[k] * inv).T


def _hog_pool(x_ref, as_ref, ad_ref, pm_ref, pmt_ref, ch):
    xb = x_ref[ch].astype(jnp.bfloat16)

    s = jnp.dot(as_ref[...], xb, preferred_element_type=jnp.float32)
    d = jnp.dot(ad_ref[...], xb, preferred_element_type=jnp.float32)

    sl = jnp.concatenate([s[:, 1:2], s[:, :-1]], axis=1)
    sr = jnp.concatenate([s[:, 1:], s[:, -2:-1]], axis=1)
    dl = jnp.concatenate([d[:, 1:2], d[:, :-1]], axis=1)
    dr = jnp.concatenate([d[:, 1:], d[:, -2:-1]], axis=1)
    gx = sl - sr
    gy = dl + 2.0 * d + dr

    norm = jnp.sqrt(gx * gx + gy * gy)

    neg = (gx < 0.0) | ((gx == 0.0) & (gy < 0.0))
    u = jnp.where(neg, -gy, gy)
    v = jnp.where(neg, -gx, gx)

    pm = pm_ref[...]
    cols = [jnp.dot(norm, pm, preferred_element_type=jnp.float32).T]
    for k in range(1, _NBINS):
        if _COS[k] > 0.0:
            mask = v >= u * _TAN[k]
        else:
            mask = v <= u * _TAN[k]
        t_k = jnp.where(mask, norm, 0.0)
        cols.append(jnp.dot(t_k, pm, preferred_element_type=jnp.float32).T)
    stacked = jnp.concatenate(cols, axis=0)

    return jnp.dot(stacked, pm_ref[...].astype(jnp.float32),
                   preferred_element_type=jnp.float32)


_IMGS = 3


@jax.jit
def kernel(x):
    b, c, h, w = x.shape
    n = b * c
    xr = x.reshape(n, h, w)
    a_s, a_d = _band_matrices()
    pm = _pool_matrix()
    pmt = pm.T.astype(jnp.float32)
    out = pl.pallas_call(
        _hog_body,
        grid=(n // _IMGS,),
        in_specs=[
            pl.BlockSpec((_IMGS, h, w), lambda i: (i, 0, 0)),
            pl.BlockSpec((_H, _H), lambda i: (0, 0)),
            pl.BlockSpec((_H, _H), lambda i: (0, 0)),
            pl.BlockSpec((_W, _WC), lambda i: (0, 0)),
            pl.BlockSpec((_HC, _H), lambda i: (0, 0)),
        ],
        out_specs=pl.BlockSpec((_IMGS, _NBINS, _HC, _WC),
                               lambda i: (i, 0, 0, 0)),
        out_shape=jax.ShapeDtypeStruct((n, _NBINS, _HC, _WC), jnp.float32),
    )(xr, a_s, a_d, pm, pmt)
    return out.reshape(b, c, _NBINS, _HC, _WC)
